# baseline (speedup 1.0000x reference)
import jax
import jax.numpy as jnp
from jax.experimental import pallas as pl
from jax.experimental.pallas import tpu as pltpu


def _sage(x_src, x_dst, ei, Wl, bl, Wr):
    msg = jnp.take(x_src, ei[0], axis=0)
    num_dst = x_dst.shape[0]
    summ = jax.ops.segment_sum(msg, ei[1], num_segments=num_dst)
    cnt = jax.ops.segment_sum(jnp.ones((ei.shape[1],), msg.dtype), ei[1], num_segments=num_dst)
    mean = summ / jnp.clip(cnt, 1.0)[:, None]
    return mean @ Wl + bl + x_dst @ Wr


def _gate_body(nu_ref, ni_ref, fu_ref, cu_ref, iu_ref, fi_ref, ci_ref, ii_ref,
               ou_ref, oi_ref):
    ou_ref[...] = fu_ref[...] * cu_ref[...] + iu_ref[...] * jnp.tanh(nu_ref[...])
    oi_ref[...] = fi_ref[...] * ci_ref[...] + ii_ref[...] * jnp.tanh(ni_ref[...])


def kernel(x_user, x_item, edge_index_user_item, edge_index_item_user,
           h_user, h_item, c_user, c_item, i_user, i_item, f_user, f_item,
           Wl1_ui, bl1_ui, Wr1_ui, Wl1_iu, bl1_iu, Wr1_iu,
           Wl2_ui, bl2_ui, Wr2_ui, Wl2_iu, bl2_iu, Wr2_iu):
    t_u, t_i = x_user, x_item
    ni = _sage(t_u, t_i, edge_index_user_item, Wl1_ui, bl1_ui, Wr1_ui)
    nu = _sage(t_i, t_u, edge_index_item_user, Wl1_iu, bl1_iu, Wr1_iu)
    t_u, t_i = nu, ni
    ni = _sage(t_u, t_i, edge_index_user_item, Wl2_ui, bl2_ui, Wr2_ui)
    nu = _sage(t_i, t_u, edge_index_item_user, Wl2_iu, bl2_iu, Wr2_iu)
    N, D = nu.shape
    out_u, out_i = pl.pallas_call(
        _gate_body,
        out_shape=(jax.ShapeDtypeStruct((N, D), jnp.float32),
                   jax.ShapeDtypeStruct((N, D), jnp.float32)),
    )(nu, ni, f_user, c_user, i_user, f_item, c_item, i_item)
    return out_u, out_i


# speedup vs baseline: 5.0228x; 5.0228x over previous
import functools

import jax
import jax.numpy as jnp
from jax import lax
from jax.experimental import pallas as pl
from jax.experimental.pallas import tpu as pltpu
from jax.experimental.pallas import tpu_sc as plsc

N = 10000
D = 128
E = 320000

NC = 2
NS = 16
B = 128
NBW = 160
NBT = NS * NBW
E_PAD = NBT * B
N_PAD = 10240
RPT = N_PAD // NS
CW = 16

_mesh = plsc.VectorSubcoreMesh(core_axis_name="c", subcore_axis_name="s",
                               num_cores=NC, num_subcores=NS)


IC = 8
NCH = NBW // IC


def _agg_body(table, srcb, dstb, summ_out,
              summ_acc, src_c, dst_c, rows_v, sem):
    c = lax.axis_index("c")
    s = lax.axis_index("s")
    base = s * NBW
    def _fill_zrow(i, carry):
        for k in range(D // 16):
            rows_v[i, pl.ds(k * 16, 16)] = jnp.zeros((16,), jnp.float32)
        return carry
    lax.fori_loop(0, B, _fill_zrow, 0)
    r0 = s * RPT
    for k in range(RPT // B):
        pltpu.sync_copy(rows_v, summ_acc.at[pl.ds(r0 + k * B, B)])
    plsc.subcore_barrier()

    def _chunk(ch, carry):
        cb = base + ch * IC
        pltpu.sync_copy(srcb.at[c, pl.ds(cb, IC)], src_c)
        pltpu.sync_copy(dstb.at[c, pl.ds(cb, IC)], dst_c)
        for j in range(IC):
            pltpu.async_copy(table.at[src_c.at[j]], rows_v, sem).wait()
            pltpu.sync_copy(rows_v, summ_acc.at[dst_c.at[j]], add=True)
        return carry
    lax.fori_loop(0, NCH, _chunk, 0)
    plsc.subcore_barrier()

    for k in range(RPT // B):
        pltpu.sync_copy(summ_acc.at[pl.ds(r0 + k * B, B)],
                        summ_out.at[c, pl.ds(r0 + k * B, B)])


_agg = pl.kernel(
    _agg_body,
    out_type=jax.ShapeDtypeStruct((NC, N_PAD, D), jnp.float32),
    mesh=_mesh,
    scratch_types=[
        pltpu.VMEM_SHARED((N_PAD, D), jnp.float32),
        pltpu.VMEM((IC, B), jnp.int32),
        pltpu.VMEM((IC, B), jnp.int32),
        pltpu.VMEM((B, D), jnp.float32),
        pltpu.SemaphoreType.DMA,
    ],
)


BN = 1000


def _dense1_body(s0, c0, s1, c1, xu, xi,
                 Wl_ui, bl_ui, Wr_ui, Wl_iu, bl_iu, Wr_iu,
                 nu_out, ni_out):
    mean0 = s0[...] / jnp.maximum(c0[...][:, 0:1], 1.0)
    ni_out[...] = (jnp.dot(mean0, Wl_ui[...], preferred_element_type=jnp.float32)
                   + bl_ui[...]
                   + jnp.dot(xi[...], Wr_ui[...], preferred_element_type=jnp.float32))
    mean1 = s1[...] / jnp.maximum(c1[...][:, 0:1], 1.0)
    nu_out[...] = (jnp.dot(mean1, Wl_iu[...], preferred_element_type=jnp.float32)
                   + bl_iu[...]
                   + jnp.dot(xu[...], Wr_iu[...], preferred_element_type=jnp.float32))


def _dense2_body(s0, c0, s1, c1, nu, ni,
                 Wl_ui, bl_ui, Wr_ui, Wl_iu, bl_iu, Wr_iu,
                 fu, cu, iu, fi, ci, ii,
                 ou_out, oi_out):
    mean0 = s0[...] / jnp.maximum(c0[...][:, 0:1], 1.0)
    t_i = jnp.tanh(jnp.dot(mean0, Wl_ui[...], preferred_element_type=jnp.float32)
                   + bl_ui[...]
                   + jnp.dot(ni[...], Wr_ui[...], preferred_element_type=jnp.float32))
    oi_out[...] = fi[...] * ci[...] + ii[...] * t_i
    mean1 = s1[...] / jnp.maximum(c1[...][:, 0:1], 1.0)
    t_u = jnp.tanh(jnp.dot(mean1, Wl_iu[...], preferred_element_type=jnp.float32)
                   + bl_iu[...]
                   + jnp.dot(nu[...], Wr_iu[...], preferred_element_type=jnp.float32))
    ou_out[...] = fu[...] * cu[...] + iu[...] * t_u


def _row_spec():
    return pl.BlockSpec((BN, D), lambda i: (i, 0))


def _cnt_spec():
    return pl.BlockSpec((BN, CW), lambda i: (i, 0))


def _w_spec():
    return pl.BlockSpec((D, D), lambda i: (0, 0))


def _b_spec():
    return pl.BlockSpec((1, D), lambda i: (0, 0))


_G = N // BN

_dense1 = pl.pallas_call(
    _dense1_body,
    grid=(_G,),
    in_specs=[_row_spec(), _cnt_spec(), _row_spec(), _cnt_spec(),
              _row_spec(), _row_spec(),
              _w_spec(), _b_spec(), _w_spec(), _w_spec(), _b_spec(), _w_spec()],
    out_specs=(_row_spec(), _row_spec()),
    out_shape=(jax.ShapeDtypeStruct((N, D), jnp.float32),
               jax.ShapeDtypeStruct((N, D), jnp.float32)),
)

_dense2 = pl.pallas_call(
    _dense2_body,
    grid=(_G,),
    in_specs=[_row_spec(), _cnt_spec(), _row_spec(), _cnt_spec(),
              _row_spec(), _row_spec(),
              _w_spec(), _b_spec(), _w_spec(), _w_spec(), _b_spec(), _w_spec(),
              _row_spec(), _row_spec(), _row_spec(),
              _row_spec(), _row_spec(), _row_spec()],
    out_specs=(_row_spec(), _row_spec()),
    out_shape=(jax.ShapeDtypeStruct((N, D), jnp.float32),
               jax.ShapeDtypeStruct((N, D), jnp.float32)),
)


def _prep_idx(ei, src_offset):
    src = ei[0].astype(jnp.int32)
    dst = ei[1].astype(jnp.int32)
    pad = E_PAD - src.shape[0]
    ar = jnp.arange(pad, dtype=jnp.int32)
    src = jnp.concatenate([src, ar % N]) + src_offset
    dst = jnp.concatenate([dst, N + ar % (N_PAD - N)])
    return src.reshape(NBT, B), dst.reshape(NBT, B)


def kernel(x_user, x_item, edge_index_user_item, edge_index_item_user,
           h_user, h_item, c_user, c_item, i_user, i_item, f_user, f_item,
           Wl1_ui, bl1_ui, Wr1_ui, Wl1_iu, bl1_iu, Wr1_iu,
           Wl2_ui, bl2_ui, Wr2_ui, Wl2_iu, bl2_iu, Wr2_iu):
    src_ui, dst_ui = _prep_idx(edge_index_user_item, 0)
    src_iu, dst_iu = _prep_idx(edge_index_item_user, N)
    srcb = jnp.stack([src_ui, src_iu])
    dstb = jnp.stack([dst_ui, dst_iu])

    cnt = _agg(jnp.ones((2 * N, D), jnp.float32), srcb, dstb)
    table1 = jnp.concatenate([x_user, x_item], axis=0)
    summ1 = _agg(table1, srcb, dstb)
    c_ui = cnt[0, :N, :CW]
    c_iu = cnt[1, :N, :CW]

    nu, ni = _dense1(summ1[0, :N], c_ui, summ1[1, :N], c_iu, x_user, x_item,
                     Wl1_ui, bl1_ui.reshape(1, D), Wr1_ui,
                     Wl1_iu, bl1_iu.reshape(1, D), Wr1_iu)

    table2 = jnp.concatenate([nu, ni], axis=0)
    summ2 = _agg(table2, srcb, dstb)

    out_u, out_i = _dense2(summ2[0, :N], c_ui, summ2[1, :N], c_iu, nu, ni,
                           Wl2_ui, bl2_ui.reshape(1, D), Wr2_ui,
                           Wl2_iu, bl2_iu.reshape(1, D), Wr2_iu,
                           f_user, c_user, i_user, f_item, c_item, i_item)
    return out_u, out_i


# speedup vs baseline: 6.9982x; 1.3933x over previous
import functools

import jax
import jax.numpy as jnp
from jax import lax
from jax.experimental import pallas as pl
from jax.experimental.pallas import tpu as pltpu
from jax.experimental.pallas import tpu_sc as plsc

N = 10000
D = 128
E = 320000

NC = 2
NS = 16
B = 128
NBW = 160
NBT = NS * NBW
E_PAD = NBT * B
N_PAD = 10240
RPT = N_PAD // NS
CW = 16

_mesh = plsc.VectorSubcoreMesh(core_axis_name="c", subcore_axis_name="s",
                               num_cores=NC, num_subcores=NS)


IC = 8
NCH = NBW // IC


def _agg_body(table, srcb, dstb, summ_out,
              summ_acc, src_c, dst_c, rows0, rows1, sem0, sem1):
    c = lax.axis_index("c")
    s = lax.axis_index("s")
    base = s * NBW
    def _fill_zrow(i, carry):
        for k in range(D // 16):
            rows0[i, pl.ds(k * 16, 16)] = jnp.zeros((16,), jnp.float32)
        return carry
    lax.fori_loop(0, B, _fill_zrow, 0)
    r0 = s * RPT
    for k in range(RPT // B):
        pltpu.sync_copy(rows0, summ_acc.at[pl.ds(r0 + k * B, B)])
    pltpu.sync_copy(srcb.at[c, pl.ds(base, IC)], src_c)
    pltpu.sync_copy(dstb.at[c, pl.ds(base, IC)], dst_c)
    pltpu.async_copy(table.at[src_c.at[0]], rows0, sem0)
    plsc.subcore_barrier()

    rows = (rows0, rows1)
    sems = (sem0, sem1)

    def _chunk(ch, carry):
        for j in range(IC):
            b = j % 2
            nb = (j + 1) % 2
            if j < IC - 1:
                pltpu.async_copy(table.at[src_c.at[j + 1]], rows[nb], sems[nb])
            pltpu.make_async_copy(table.at[pl.ds(0, B)], rows[b], sems[b]).wait()
            pltpu.sync_copy(rows[b], summ_acc.at[dst_c.at[j]], add=True)

        @pl.when(ch + 1 < NCH)
        def _():
            cb = base + (ch + 1) * IC
            pltpu.sync_copy(srcb.at[c, pl.ds(cb, IC)], src_c)
            pltpu.sync_copy(dstb.at[c, pl.ds(cb, IC)], dst_c)
            pltpu.async_copy(table.at[src_c.at[0]], rows0, sem0)
        return carry
    lax.fori_loop(0, NCH, _chunk, 0)
    plsc.subcore_barrier()

    for k in range(RPT // B):
        pltpu.sync_copy(summ_acc.at[pl.ds(r0 + k * B, B)],
                        summ_out.at[c, pl.ds(r0 + k * B, B)])


_agg = pl.kernel(
    _agg_body,
    out_type=jax.ShapeDtypeStruct((NC, N_PAD, D), jnp.float32),
    mesh=_mesh,
    scratch_types=[
        pltpu.VMEM_SHARED((N_PAD, D), jnp.float32),
        pltpu.VMEM((IC, B), jnp.int32),
        pltpu.VMEM((IC, B), jnp.int32),
        pltpu.VMEM((B, D), jnp.float32),
        pltpu.VMEM((B, D), jnp.float32),
        pltpu.SemaphoreType.DMA,
        pltpu.SemaphoreType.DMA,
    ],
)


BN = 1000


def _dense1_body(s0, c0, s1, c1, xu, xi,
                 Wl_ui, bl_ui, Wr_ui, Wl_iu, bl_iu, Wr_iu,
                 nu_out, ni_out):
    mean0 = s0[...] / jnp.maximum(c0[...][:, 0:1], 1.0)
    ni_out[...] = (jnp.dot(mean0, Wl_ui[...], preferred_element_type=jnp.float32)
                   + bl_ui[...]
                   + jnp.dot(xi[...], Wr_ui[...], preferred_element_type=jnp.float32))
    mean1 = s1[...] / jnp.maximum(c1[...][:, 0:1], 1.0)
    nu_out[...] = (jnp.dot(mean1, Wl_iu[...], preferred_element_type=jnp.float32)
                   + bl_iu[...]
                   + jnp.dot(xu[...], Wr_iu[...], preferred_element_type=jnp.float32))


def _dense2_body(s0, c0, s1, c1, nu, ni,
                 Wl_ui, bl_ui, Wr_ui, Wl_iu, bl_iu, Wr_iu,
                 fu, cu, iu, fi, ci, ii,
                 ou_out, oi_out):
    mean0 = s0[...] / jnp.maximum(c0[...][:, 0:1], 1.0)
    t_i = jnp.tanh(jnp.dot(mean0, Wl_ui[...], preferred_element_type=jnp.float32)
                   + bl_ui[...]
                   + jnp.dot(ni[...], Wr_ui[...], preferred_element_type=jnp.float32))
    oi_out[...] = fi[...] * ci[...] + ii[...] * t_i
    mean1 = s1[...] / jnp.maximum(c1[...][:, 0:1], 1.0)
    t_u = jnp.tanh(jnp.dot(mean1, Wl_iu[...], preferred_element_type=jnp.float32)
                   + bl_iu[...]
                   + jnp.dot(nu[...], Wr_iu[...], preferred_element_type=jnp.float32))
    ou_out[...] = fu[...] * cu[...] + iu[...] * t_u


def _row_spec():
    return pl.BlockSpec((BN, D), lambda i: (i, 0))


def _cnt_spec():
    return pl.BlockSpec((BN, CW), lambda i: (i, 0))


def _w_spec():
    return pl.BlockSpec((D, D), lambda i: (0, 0))


def _b_spec():
    return pl.BlockSpec((1, D), lambda i: (0, 0))


_G = N // BN

_dense1 = pl.pallas_call(
    _dense1_body,
    grid=(_G,),
    in_specs=[_row_spec(), _cnt_spec(), _row_spec(), _cnt_spec(),
              _row_spec(), _row_spec(),
              _w_spec(), _b_spec(), _w_spec(), _w_spec(), _b_spec(), _w_spec()],
    out_specs=(_row_spec(), _row_spec()),
    out_shape=(jax.ShapeDtypeStruct((N, D), jnp.float32),
               jax.ShapeDtypeStruct((N, D), jnp.float32)),
)

_dense2 = pl.pallas_call(
    _dense2_body,
    grid=(_G,),
    in_specs=[_row_spec(), _cnt_spec(), _row_spec(), _cnt_spec(),
              _row_spec(), _row_spec(),
              _w_spec(), _b_spec(), _w_spec(), _w_spec(), _b_spec(), _w_spec(),
              _row_spec(), _row_spec(), _row_spec(),
              _row_spec(), _row_spec(), _row_spec()],
    out_specs=(_row_spec(), _row_spec()),
    out_shape=(jax.ShapeDtypeStruct((N, D), jnp.float32),
               jax.ShapeDtypeStruct((N, D), jnp.float32)),
)


def _prep_idx(ei, src_offset):
    src = ei[0].astype(jnp.int32)
    dst = ei[1].astype(jnp.int32)
    pad = E_PAD - src.shape[0]
    ar = jnp.arange(pad, dtype=jnp.int32)
    src = jnp.concatenate([src, ar % N]) + src_offset
    dst = jnp.concatenate([dst, N + ar % (N_PAD - N)])
    return src.reshape(NBT, B), dst.reshape(NBT, B)


def kernel(x_user, x_item, edge_index_user_item, edge_index_item_user,
           h_user, h_item, c_user, c_item, i_user, i_item, f_user, f_item,
           Wl1_ui, bl1_ui, Wr1_ui, Wl1_iu, bl1_iu, Wr1_iu,
           Wl2_ui, bl2_ui, Wr2_ui, Wl2_iu, bl2_iu, Wr2_iu):
    src_ui, dst_ui = _prep_idx(edge_index_user_item, 0)
    src_iu, dst_iu = _prep_idx(edge_index_item_user, N)
    srcb = jnp.stack([src_ui, src_iu])
    dstb = jnp.stack([dst_ui, dst_iu])

    cnt = _agg(jnp.ones((2 * N, D), jnp.float32), srcb, dstb)
    table1 = jnp.concatenate([x_user, x_item], axis=0)
    summ1 = _agg(table1, srcb, dstb)
    c_ui = cnt[0, :N, :CW]
    c_iu = cnt[1, :N, :CW]

    nu, ni = _dense1(summ1[0, :N], c_ui, summ1[1, :N], c_iu, x_user, x_item,
                     Wl1_ui, bl1_ui.reshape(1, D), Wr1_ui,
                     Wl1_iu, bl1_iu.reshape(1, D), Wr1_iu)

    table2 = jnp.concatenate([nu, ni], axis=0)
    summ2 = _agg(table2, srcb, dstb)

    out_u, out_i = _dense2(summ2[0, :N], c_ui, summ2[1, :N], c_iu, nu, ni,
                           Wl2_ui, bl2_ui.reshape(1, D), Wr2_ui,
                           Wl2_iu, bl2_iu.reshape(1, D), Wr2_iu,
                           f_user, c_user, i_user, f_item, c_item, i_item)
    return out_u, out_i


# speedup vs baseline: 8.7141x; 1.2452x over previous
import jax
import jax.numpy as jnp
from jax import lax
from jax.experimental import pallas as pl
from jax.experimental.pallas import tpu as pltpu
from jax.experimental.pallas import tpu_sc as plsc

N = 10000
D = 128
E = 320000

NC = 2
NS = 16
B = 128
NBW = 160
NBT = NS * NBW
E_PAD = NBT * B
N_PAD = 10240
RPT = N_PAD // NS
IC = 16
NCH = NBW // IC

_mesh = plsc.VectorSubcoreMesh(core_axis_name="c", subcore_axis_name="s",
                               num_cores=NC, num_subcores=NS)


def _agg_body(table, srcb, dstb, summ_out, summ_acc,
              srcA, dstA, srcB, dstB, rows0, rows1,
              gsem0, gsem1, isemA, isemB):
    c = lax.axis_index("c")
    s = lax.axis_index("s")
    base = s * NBW
    r0 = s * RPT
    rows = (rows0, rows1)
    gsems = (gsem0, gsem1)

    def _fill_zrow(i, carry):
        for k in range(D // 16):
            z = jnp.zeros((16,), jnp.float32)
            rows0[i, pl.ds(k * 16, 16)] = z
            rows1[i, pl.ds(k * 16, 16)] = z
        return carry
    lax.fori_loop(0, B, _fill_zrow, 0)
    for k in range(RPT // B):
        pltpu.sync_copy(rows0, summ_acc.at[pl.ds(r0 + k * B, B)])
    pltpu.sync_copy(srcb.at[c, pl.ds(base, IC)], srcA)
    pltpu.sync_copy(dstb.at[c, pl.ds(base, IC)], dstA)
    pltpu.async_copy(table.at[srcA.at[0]], rows0, gsem0)
    plsc.subcore_barrier()

    def _do_chunk(ch, srcX, dstX, srcY, dstY, isemY, more):
        for j in range(IC):
            b = j % 2
            nb = (j + 1) % 2
            if j == 0:
                def _prefetch():
                    cb = base + (ch + 1) * IC
                    pltpu.async_copy(srcb.at[c, pl.ds(cb, IC)], srcY, isemY)
                    pltpu.async_copy(dstb.at[c, pl.ds(cb, IC)], dstY, isemY)
                if more is True:
                    _prefetch()
                else:
                    pl.when(more)(_prefetch)
            if j < IC - 1:
                pltpu.async_copy(table.at[srcX.at[j + 1]], rows[nb], gsems[nb])
            else:
                def _next_gather():
                    cb = base + (ch + 1) * IC
                    pltpu.make_async_copy(srcb.at[c, pl.ds(cb, IC)], srcY,
                                          isemY).wait()
                    pltpu.make_async_copy(dstb.at[c, pl.ds(cb, IC)], dstY,
                                          isemY).wait()
                    pltpu.async_copy(table.at[srcY.at[0]], rows[nb], gsems[nb])
                if more is True:
                    _next_gather()
                else:
                    pl.when(more)(_next_gather)
            pltpu.make_async_copy(table.at[pl.ds(0, B)], rows[b],
                                  gsems[b]).wait()
            pltpu.sync_copy(rows[b], summ_acc.at[dstX.at[j]], add=True)

    def _pair(u, carry):
        _do_chunk(2 * u, srcA, dstA, srcB, dstB, isemB, True)
        _do_chunk(2 * u + 1, srcB, dstB, srcA, dstA, isemA, u + 1 < NCH // 2)
        return carry
    lax.fori_loop(0, NCH // 2, _pair, 0)
    plsc.subcore_barrier()

    for k in range(RPT // B):
        pltpu.sync_copy(summ_acc.at[pl.ds(r0 + k * B, B)],
                        summ_out.at[c, pl.ds(r0 + k * B, B)])


_agg = pl.kernel(
    _agg_body,
    out_type=jax.ShapeDtypeStruct((NC, N_PAD, D), jnp.float32),
    mesh=_mesh,
    scratch_types=[
        pltpu.VMEM_SHARED((N_PAD, D), jnp.float32),
        pltpu.VMEM((IC, B), jnp.int32),
        pltpu.VMEM((IC, B), jnp.int32),
        pltpu.VMEM((IC, B), jnp.int32),
        pltpu.VMEM((IC, B), jnp.int32),
        pltpu.VMEM((B, D), jnp.float32),
        pltpu.VMEM((B, D), jnp.float32),
        pltpu.SemaphoreType.DMA,
        pltpu.SemaphoreType.DMA,
        pltpu.SemaphoreType.DMA,
        pltpu.SemaphoreType.DMA,
    ],
)


BN = 1000
_G = N // BN


def _dense1_body(s01, cnt, xu, xi,
                 Wl_ui, bl_ui, Wr_ui, Wl_iu, bl_iu, Wr_iu, t2_out):
    mean0 = s01[0] / jnp.maximum(cnt[0][:, 0:1], 1.0)
    t2_out[1, :, :] = (jnp.dot(mean0, Wl_ui[...],
                               preferred_element_type=jnp.float32)
                       + bl_ui[...]
                       + jnp.dot(xi[...], Wr_ui[...],
                                 preferred_element_type=jnp.float32))
    mean1 = s01[1] / jnp.maximum(cnt[1][:, 0:1], 1.0)
    t2_out[0, :, :] = (jnp.dot(mean1, Wl_iu[...],
                               preferred_element_type=jnp.float32)
                       + bl_iu[...]
                       + jnp.dot(xu[...], Wr_iu[...],
                                 preferred_element_type=jnp.float32))


def _dense2_body(s01, cnt, t2,
                 Wl_ui, bl_ui, Wr_ui, Wl_iu, bl_iu, Wr_iu,
                 fu, cu, iu, fi, ci, ii,
                 ou_out, oi_out):
    mean0 = s01[0] / jnp.maximum(cnt[0][:, 0:1], 1.0)
    t_i = jnp.tanh(jnp.dot(mean0, Wl_ui[...],
                           preferred_element_type=jnp.float32)
                   + bl_ui[...]
                   + jnp.dot(t2[1], Wr_ui[...],
                             preferred_element_type=jnp.float32))
    oi_out[...] = fi[...] * ci[...] + ii[...] * t_i
    mean1 = s01[1] / jnp.maximum(cnt[1][:, 0:1], 1.0)
    t_u = jnp.tanh(jnp.dot(mean1, Wl_iu[...],
                           preferred_element_type=jnp.float32)
                   + bl_iu[...]
                   + jnp.dot(t2[0], Wr_iu[...],
                             preferred_element_type=jnp.float32))
    ou_out[...] = fu[...] * cu[...] + iu[...] * t_u


def _pad_spec():
    return pl.BlockSpec((NC, BN, D), lambda i: (0, i, 0))


def _row_spec():
    return pl.BlockSpec((BN, D), lambda i: (i, 0))


def _w_spec():
    return pl.BlockSpec((D, D), lambda i: (0, 0))


def _b_spec():
    return pl.BlockSpec((1, D), lambda i: (0, 0))


_dense1 = pl.pallas_call(
    _dense1_body,
    grid=(_G,),
    in_specs=[_pad_spec(), _pad_spec(), _row_spec(), _row_spec(),
              _w_spec(), _b_spec(), _w_spec(), _w_spec(), _b_spec(), _w_spec()],
    out_specs=pl.BlockSpec((NC, BN, D), lambda i: (0, i, 0)),
    out_shape=jax.ShapeDtypeStruct((NC, N, D), jnp.float32),
)

_dense2 = pl.pallas_call(
    _dense2_body,
    grid=(_G,),
    in_specs=[_pad_spec(), _pad_spec(),
              pl.BlockSpec((NC, BN, D), lambda i: (0, i, 0)),
              _w_spec(), _b_spec(), _w_spec(), _w_spec(), _b_spec(), _w_spec(),
              _row_spec(), _row_spec(), _row_spec(),
              _row_spec(), _row_spec(), _row_spec()],
    out_specs=(_row_spec(), _row_spec()),
    out_shape=(jax.ShapeDtypeStruct((N, D), jnp.float32),
               jax.ShapeDtypeStruct((N, D), jnp.float32)),
)


def _prep_idx(ei, src_offset):
    src = ei[0].astype(jnp.int32)
    dst = ei[1].astype(jnp.int32)
    pad = E_PAD - src.shape[0]
    ar = jnp.arange(pad, dtype=jnp.int32)
    src = jnp.concatenate([src, ar % N]) + src_offset
    dst = jnp.concatenate([dst, N + ar % (N_PAD - N)])
    return src.reshape(NBT, B), dst.reshape(NBT, B)


def kernel(x_user, x_item, edge_index_user_item, edge_index_item_user,
           h_user, h_item, c_user, c_item, i_user, i_item, f_user, f_item,
           Wl1_ui, bl1_ui, Wr1_ui, Wl1_iu, bl1_iu, Wr1_iu,
           Wl2_ui, bl2_ui, Wr2_ui, Wl2_iu, bl2_iu, Wr2_iu):
    src_ui, dst_ui = _prep_idx(edge_index_user_item, 0)
    src_iu, dst_iu = _prep_idx(edge_index_item_user, N)
    srcb = jnp.stack([src_ui, src_iu])
    dstb = jnp.stack([dst_ui, dst_iu])

    cnt = _agg(jnp.ones((2 * N, D), jnp.float32), srcb, dstb)
    table1 = jnp.concatenate([x_user, x_item], axis=0)
    summ1 = _agg(table1, srcb, dstb)

    t2 = _dense1(summ1, cnt, x_user, x_item,
                 Wl1_ui, bl1_ui.reshape(1, D), Wr1_ui,
                 Wl1_iu, bl1_iu.reshape(1, D), Wr1_iu)

    summ2 = _agg(t2.reshape(2 * N, D), srcb, dstb)

    out_u, out_i = _dense2(summ2, cnt, t2,
                           Wl2_ui, bl2_ui.reshape(1, D), Wr2_ui,
                           Wl2_iu, bl2_iu.reshape(1, D), Wr2_iu,
                           f_user, c_user, i_user, f_item, c_item, i_item)
    return out_u, out_i


# speedup vs baseline: 9.5740x; 1.0987x over previous
import jax
import jax.numpy as jnp
from jax import lax
from jax.experimental import pallas as pl
from jax.experimental.pallas import tpu as pltpu
from jax.experimental.pallas import tpu_sc as plsc

N = 10000
D = 128
E = 320000

NC = 2
NS = 16
B = 128
NBW = 160
NBT = NS * NBW
E_PAD = NBT * B
N_PAD = 10240
RPT = N_PAD // NS
IC = 16
NCH = NBW // IC

_mesh = plsc.VectorSubcoreMesh(core_axis_name="c", subcore_axis_name="s",
                               num_cores=NC, num_subcores=NS)


def _agg_body(table, srcb, dstb, summ_out, summ_acc,
              srcA, dstA, srcB, dstB, rows0, rows1,
              gsem0, gsem1, isemA, isemB):
    c = lax.axis_index("c")
    s = lax.axis_index("s")
    base = s * NBW
    r0 = s * RPT
    rows = (rows0, rows1)
    gsems = (gsem0, gsem1)

    def _fill_zrow(i, carry):
        for k in range(D // 16):
            z = jnp.zeros((16,), jnp.float32)
            rows0[i, pl.ds(k * 16, 16)] = z
            rows1[i, pl.ds(k * 16, 16)] = z
        return carry
    lax.fori_loop(0, B, _fill_zrow, 0)
    for k in range(RPT // B):
        pltpu.sync_copy(rows0, summ_acc.at[pl.ds(r0 + k * B, B)])
    pltpu.sync_copy(srcb.at[c, pl.ds(base, IC)], srcA)
    pltpu.sync_copy(dstb.at[c, pl.ds(base, IC)], dstA)
    pltpu.async_copy(table.at[srcA.at[0]], rows0, gsem0)
    plsc.subcore_barrier()

    def _do_chunk(ch, srcX, dstX, srcY, dstY, isemY, more):
        for j in range(IC):
            b = j % 2
            nb = (j + 1) % 2
            if j == 0:
                def _prefetch():
                    cb = base + (ch + 1) * IC
                    pltpu.async_copy(srcb.at[c, pl.ds(cb, IC)], srcY, isemY)
                    pltpu.async_copy(dstb.at[c, pl.ds(cb, IC)], dstY, isemY)
                if more is True:
                    _prefetch()
                else:
                    pl.when(more)(_prefetch)
            if j < IC - 1:
                pltpu.async_copy(table.at[srcX.at[j + 1]], rows[nb], gsems[nb])
            else:
                def _next_gather():
                    cb = base + (ch + 1) * IC
                    pltpu.make_async_copy(srcb.at[c, pl.ds(cb, IC)], srcY,
                                          isemY).wait()
                    pltpu.make_async_copy(dstb.at[c, pl.ds(cb, IC)], dstY,
                                          isemY).wait()
                    pltpu.async_copy(table.at[srcY.at[0]], rows[nb], gsems[nb])
                if more is True:
                    _next_gather()
                else:
                    pl.when(more)(_next_gather)
            pltpu.make_async_copy(table.at[pl.ds(0, B)], rows[b],
                                  gsems[b]).wait()
            pltpu.sync_copy(rows[b], summ_acc.at[dstX.at[j]], add=True)

    def _pair(u, carry):
        _do_chunk(2 * u, srcA, dstA, srcB, dstB, isemB, True)
        _do_chunk(2 * u + 1, srcB, dstB, srcA, dstA, isemA, u + 1 < NCH // 2)
        return carry
    lax.fori_loop(0, NCH // 2, _pair, 0)
    plsc.subcore_barrier()

    for k in range(RPT // B):
        pltpu.sync_copy(summ_acc.at[pl.ds(r0 + k * B, B)],
                        summ_out.at[c, pl.ds(r0 + k * B, B)])


_agg = pl.kernel(
    _agg_body,
    out_type=jax.ShapeDtypeStruct((NC, N_PAD, D), jnp.float32),
    mesh=_mesh,
    scratch_types=[
        pltpu.VMEM_SHARED((N_PAD, D), jnp.float32),
        pltpu.VMEM((IC, B), jnp.int32),
        pltpu.VMEM((IC, B), jnp.int32),
        pltpu.VMEM((IC, B), jnp.int32),
        pltpu.VMEM((IC, B), jnp.int32),
        pltpu.VMEM((B, D), jnp.float32),
        pltpu.VMEM((B, D), jnp.float32),
        pltpu.SemaphoreType.DMA,
        pltpu.SemaphoreType.DMA,
        pltpu.SemaphoreType.DMA,
        pltpu.SemaphoreType.DMA,
    ],
)


def _cnt_body(dstb, cnt_out, cnt_acc, dstA, dstB, rows0, ones_v,
              isemA, isemB):
    c = lax.axis_index("c")
    s = lax.axis_index("s")
    base = s * NBW
    r0 = s * RPT

    def _fill(i, carry):
        for k in range(D // 16):
            rows0[i, pl.ds(k * 16, 16)] = jnp.zeros((16,), jnp.float32)
            ones_v[i, pl.ds(k * 16, 16)] = jnp.ones((16,), jnp.float32)
        return carry
    lax.fori_loop(0, B, _fill, 0)
    for k in range(RPT // B):
        pltpu.sync_copy(rows0, cnt_acc.at[pl.ds(r0 + k * B, B)])
    pltpu.sync_copy(dstb.at[c, pl.ds(base, IC)], dstA)
    plsc.subcore_barrier()

    def _do_chunk(ch, dstX, dstY, isemY, more):
        for j in range(IC):
            if j == 0:
                def _prefetch():
                    cb = base + (ch + 1) * IC
                    pltpu.async_copy(dstb.at[c, pl.ds(cb, IC)], dstY, isemY)
                if more is True:
                    _prefetch()
                else:
                    pl.when(more)(_prefetch)
            if j == IC - 1:
                def _wait_idx():
                    cb = base + (ch + 1) * IC
                    pltpu.make_async_copy(dstb.at[c, pl.ds(cb, IC)], dstY,
                                          isemY).wait()
                if more is True:
                    _wait_idx()
                else:
                    pl.when(more)(_wait_idx)
            pltpu.sync_copy(ones_v, cnt_acc.at[dstX.at[j]], add=True)

    def _pair(u, carry):
        _do_chunk(2 * u, dstA, dstB, isemB, True)
        _do_chunk(2 * u + 1, dstB, dstA, isemA, u + 1 < NCH // 2)
        return carry
    lax.fori_loop(0, NCH // 2, _pair, 0)
    plsc.subcore_barrier()

    for k in range(RPT // B):
        pltpu.sync_copy(cnt_acc.at[pl.ds(r0 + k * B, B)],
                        cnt_out.at[c, pl.ds(r0 + k * B, B)])


_cnt = pl.kernel(
    _cnt_body,
    out_type=jax.ShapeDtypeStruct((NC, N_PAD, D), jnp.float32),
    mesh=_mesh,
    scratch_types=[
        pltpu.VMEM_SHARED((N_PAD, D), jnp.float32),
        pltpu.VMEM((IC, B), jnp.int32),
        pltpu.VMEM((IC, B), jnp.int32),
        pltpu.VMEM((B, D), jnp.float32),
        pltpu.VMEM((B, D), jnp.float32),
        pltpu.SemaphoreType.DMA,
        pltpu.SemaphoreType.DMA,
    ],
)


BN = 1000
_G = N // BN


def _dense1_body(s01, cnt, xu, xi,
                 Wl_ui, bl_ui, Wr_ui, Wl_iu, bl_iu, Wr_iu, t2_out):
    mean0 = s01[0] / jnp.maximum(cnt[0][:, 0:1], 1.0)
    t2_out[1, :, :] = (jnp.dot(mean0, Wl_ui[...],
                               preferred_element_type=jnp.float32)
                       + bl_ui[...]
                       + jnp.dot(xi[...], Wr_ui[...],
                                 preferred_element_type=jnp.float32))
    mean1 = s01[1] / jnp.maximum(cnt[1][:, 0:1], 1.0)
    t2_out[0, :, :] = (jnp.dot(mean1, Wl_iu[...],
                               preferred_element_type=jnp.float32)
                       + bl_iu[...]
                       + jnp.dot(xu[...], Wr_iu[...],
                                 preferred_element_type=jnp.float32))


def _dense2_body(s01, cnt, t2,
                 Wl_ui, bl_ui, Wr_ui, Wl_iu, bl_iu, Wr_iu,
                 fu, cu, iu, fi, ci, ii,
                 ou_out, oi_out):
    mean0 = s01[0] / jnp.maximum(cnt[0][:, 0:1], 1.0)
    t_i = jnp.tanh(jnp.dot(mean0, Wl_ui[...],
                           preferred_element_type=jnp.float32)
                   + bl_ui[...]
                   + jnp.dot(t2[1], Wr_ui[...],
                             preferred_element_type=jnp.float32))
    oi_out[...] = fi[...] * ci[...] + ii[...] * t_i
    mean1 = s01[1] / jnp.maximum(cnt[1][:, 0:1], 1.0)
    t_u = jnp.tanh(jnp.dot(mean1, Wl_iu[...],
                           preferred_element_type=jnp.float32)
                   + bl_iu[...]
                   + jnp.dot(t2[0], Wr_iu[...],
                             preferred_element_type=jnp.float32))
    ou_out[...] = fu[...] * cu[...] + iu[...] * t_u


def _pad_spec():
    return pl.BlockSpec((NC, BN, D), lambda i: (0, i, 0))


def _row_spec():
    return pl.BlockSpec((BN, D), lambda i: (i, 0))


def _w_spec():
    return pl.BlockSpec((D, D), lambda i: (0, 0))


def _b_spec():
    return pl.BlockSpec((1, D), lambda i: (0, 0))


_dense1 = pl.pallas_call(
    _dense1_body,
    grid=(_G,),
    in_specs=[_pad_spec(), _pad_spec(), _row_spec(), _row_spec(),
              _w_spec(), _b_spec(), _w_spec(), _w_spec(), _b_spec(), _w_spec()],
    out_specs=pl.BlockSpec((NC, BN, D), lambda i: (0, i, 0)),
    out_shape=jax.ShapeDtypeStruct((NC, N, D), jnp.float32),
)

_dense2 = pl.pallas_call(
    _dense2_body,
    grid=(_G,),
    in_specs=[_pad_spec(), _pad_spec(),
              pl.BlockSpec((NC, BN, D), lambda i: (0, i, 0)),
              _w_spec(), _b_spec(), _w_spec(), _w_spec(), _b_spec(), _w_spec(),
              _row_spec(), _row_spec(), _row_spec(),
              _row_spec(), _row_spec(), _row_spec()],
    out_specs=(_row_spec(), _row_spec()),
    out_shape=(jax.ShapeDtypeStruct((N, D), jnp.float32),
               jax.ShapeDtypeStruct((N, D), jnp.float32)),
)


def _prep_idx(ei, src_offset):
    src = ei[0].astype(jnp.int32)
    dst = ei[1].astype(jnp.int32)
    pad = E_PAD - src.shape[0]
    ar = jnp.arange(pad, dtype=jnp.int32)
    src = jnp.concatenate([src, ar % N]) + src_offset
    dst = jnp.concatenate([dst, N + ar % (N_PAD - N)])
    return src.reshape(NBT, B), dst.reshape(NBT, B)


def kernel(x_user, x_item, edge_index_user_item, edge_index_item_user,
           h_user, h_item, c_user, c_item, i_user, i_item, f_user, f_item,
           Wl1_ui, bl1_ui, Wr1_ui, Wl1_iu, bl1_iu, Wr1_iu,
           Wl2_ui, bl2_ui, Wr2_ui, Wl2_iu, bl2_iu, Wr2_iu):
    src_ui, dst_ui = _prep_idx(edge_index_user_item, 0)
    src_iu, dst_iu = _prep_idx(edge_index_item_user, N)
    srcb = jnp.stack([src_ui, src_iu])
    dstb = jnp.stack([dst_ui, dst_iu])

    cnt = _cnt(dstb)
    table1 = jnp.concatenate([x_user, x_item], axis=0)
    summ1 = _agg(table1, srcb, dstb)

    t2 = _dense1(summ1, cnt, x_user, x_item,
                 Wl1_ui, bl1_ui.reshape(1, D), Wr1_ui,
                 Wl1_iu, bl1_iu.reshape(1, D), Wr1_iu)

    summ2 = _agg(t2.reshape(2 * N, D), srcb, dstb)

    out_u, out_i = _dense2(summ2, cnt, t2,
                           Wl2_ui, bl2_ui.reshape(1, D), Wr2_ui,
                           Wl2_iu, bl2_iu.reshape(1, D), Wr2_iu,
                           f_user, c_user, i_user, f_item, c_item, i_item)
    return out_u, out_i
